# baseline (speedup 1.0000x reference)
import functools
import math

import jax
import jax.numpy as jnp
from jax import lax
from jax.experimental import pallas as pl
from jax.experimental.pallas import tpu as pltpu
from jax.experimental.pallas import tpu_sc as plsc

D = 64
LANES = 16
NC, NS = 2, 16
NW = NC * NS
SUB = 128
NBUF = 2


@functools.lru_cache(maxsize=None)
def _build(BATCH, H, V):
    assert BATCH % (NW * SUB * NBUF) == 0
    n_btiles = BATCH // SUB
    c_per_w = n_btiles // NW
    n_blocks = H * c_per_w
    n_groups = n_blocks // NBUF
    scale = jnp.float32(math.sqrt(D))

    mesh = plsc.VectorSubcoreMesh(core_axis_name="c", subcore_axis_name="s")

    @functools.partial(
        pl.kernel,
        out_type=jax.ShapeDtypeStruct((H, D // 8, n_btiles, 8, SUB), jnp.float32),
        mesh=mesh,
        compiler_params=pltpu.CompilerParams(
            use_tc_tiling_on_sc=False, needs_layout_passes=False
        ),
        scratch_types=[
            pltpu.VMEM((H, c_per_w * SUB), jnp.int32),
            pltpu.VMEM((NBUF, SUB, D), jnp.float32),
            pltpu.VMEM((NBUF, D // 8, 8, SUB), jnp.float32),
            pltpu.SemaphoreType.DMA((NBUF,)),
            pltpu.SemaphoreType.DMA((NBUF,)),
        ],
    )
    def emb(w_hbm, xt_hbm, out_hbm, idx_v, rows_v, tout_v, gsem, osem):
        wid = lax.axis_index("s") * NC + lax.axis_index("c")

        pltpu.sync_copy(
            xt_hbm.at[:, pl.ds(wid * (c_per_w * SUB), c_per_w * SUB)], idx_v
        )

        def block_hc(bi):
            h = bi // c_per_w
            cl = bi - h * c_per_w
            return h, cl

        def fire_gather(bi, b):
            h, cl = block_hc(bi)
            pltpu.async_copy(
                w_hbm.at[idx_v.at[h, pl.ds(cl * SUB, SUB)]],
                rows_v.at[b],
                gsem.at[b],
            )

        def wait_gather(b):
            pltpu.make_async_copy(
                w_hbm.at[idx_v.at[0, pl.ds(0, SUB)]],
                rows_v.at[b],
                gsem.at[b],
            ).wait()

        def fire_scatter(bi, b):
            h, cl = block_hc(bi)
            pltpu.async_copy(
                tout_v.at[b],
                out_hbm.at[h, :, wid * c_per_w + cl],
                osem.at[b],
            )

        def wait_scatter(b):
            pltpu.make_async_copy(
                tout_v.at[b],
                out_hbm.at[0, :, 0],
                osem.at[b],
            ).wait()

        lane = jax.lax.iota(jnp.int32, LANES)

        def transpose_block(b):
            def body(d, _):
                td = d // 8
                r = d - td * 8
                dvec = jnp.full((LANES,), d, dtype=jnp.int32)
                for jb in range(SUB // LANES):
                    rvec = jb * LANES + lane
                    g = plsc.load_gather(rows_v.at[b], [rvec, dvec])
                    tout_v[b, td, r, pl.ds(jb * LANES, LANES)] = g * scale
                return 0

            lax.fori_loop(0, D, body, 0)

        for b in range(NBUF):
            fire_gather(jnp.int32(b), b)

        def group(t, _):
            for b in range(NBUF):
                bi = t * NBUF + b
                wait_gather(b)

                @pl.when(t > 0)
                def _():
                    wait_scatter(b)

                transpose_block(b)
                fire_scatter(bi, b)

                @pl.when(t < n_groups - 1)
                def _():
                    fire_gather(bi + NBUF, b)

            return 0

        lax.fori_loop(0, n_groups, group, 0)

        for b in range(NBUF):
            wait_scatter(b)

    return emb


def kernel(x, W):
    Bt, H = x.shape
    V, d = W.shape
    xt = jnp.transpose(x).astype(jnp.int32)
    out_phys = _build(Bt, H, V)(W, xt)
    out = jnp.transpose(out_phys, (2, 4, 0, 1, 3)).reshape(Bt, H, d)
    return out


# speedup vs baseline: 1.4830x; 1.4830x over previous
import functools
import math

import jax
import jax.numpy as jnp
from jax import lax
from jax.experimental import pallas as pl
from jax.experimental.pallas import tpu as pltpu
from jax.experimental.pallas import tpu_sc as plsc

D = 64
LANES = 16
NC, NS = 2, 16
NW = NC * NS
SUB = 128
NBUF = 2


@functools.lru_cache(maxsize=None)
def _build(BATCH, H, V):
    assert BATCH % (NW * SUB * NBUF) == 0
    n_btiles = BATCH // SUB
    c_per_w = n_btiles // NW
    n_blocks = H * c_per_w
    n_groups = n_blocks // NBUF
    scale = jnp.float32(math.sqrt(D))

    mesh = plsc.VectorSubcoreMesh(core_axis_name="c", subcore_axis_name="s")

    @functools.partial(
        pl.kernel,
        out_type=jax.ShapeDtypeStruct((H, D // 8, n_btiles, 8, SUB), jnp.float32),
        mesh=mesh,
        compiler_params=pltpu.CompilerParams(
            use_tc_tiling_on_sc=False, needs_layout_passes=False
        ),
        scratch_types=[
            pltpu.VMEM((H, c_per_w * SUB), jnp.int32),
            pltpu.VMEM((NBUF, SUB, D), jnp.float32),
            pltpu.VMEM((NBUF, D // 8, 8, SUB), jnp.float32),
            pltpu.SemaphoreType.DMA((NBUF,)),
            pltpu.SemaphoreType.DMA((NBUF,)),
        ],
    )
    def emb(w_hbm, xt_hbm, out_hbm, idx_v, rows_v, tout_v, gsem, osem):
        wid = lax.axis_index("s") * NC + lax.axis_index("c")

        pltpu.sync_copy(
            xt_hbm.at[:, pl.ds(wid * (c_per_w * SUB), c_per_w * SUB)], idx_v
        )

        def block_hc(bi):
            h = bi // c_per_w
            cl = bi - h * c_per_w
            return h, cl

        def fire_gather(bi, b):
            h, cl = block_hc(bi)
            pltpu.async_copy(
                w_hbm.at[idx_v.at[h, pl.ds(cl * SUB, SUB)]],
                rows_v.at[b],
                gsem.at[b],
            )

        def wait_gather(b):
            pltpu.make_async_copy(
                w_hbm.at[idx_v.at[0, pl.ds(0, SUB)]],
                rows_v.at[b],
                gsem.at[b],
            ).wait()

        def fire_scatter(bi, b):
            h, cl = block_hc(bi)
            pltpu.async_copy(
                tout_v.at[b],
                out_hbm.at[h, :, wid * c_per_w + cl],
                osem.at[b],
            )

        def wait_scatter(b):
            pltpu.make_async_copy(
                tout_v.at[b],
                out_hbm.at[0, :, 0],
                osem.at[b],
            ).wait()

        lane = jax.lax.iota(jnp.int32, LANES)

        def transpose_block(b):
            @plsc.parallel_loop(0, D, unroll=8)
            def _(d):
                td = d // 8
                r = d - td * 8
                dvec = jnp.full((LANES,), d, dtype=jnp.int32)
                for jb in range(SUB // LANES):
                    rvec = jb * LANES + lane
                    g = plsc.load_gather(rows_v.at[b], [rvec, dvec])
                    tout_v[b, td, r, pl.ds(jb * LANES, LANES)] = g * scale

        for b in range(NBUF):
            fire_gather(jnp.int32(b), b)

        def group(t, _):
            for b in range(NBUF):
                bi = t * NBUF + b
                wait_gather(b)

                @pl.when(t > 0)
                def _():
                    wait_scatter(b)

                transpose_block(b)
                fire_scatter(bi, b)

                @pl.when(t < n_groups - 1)
                def _():
                    fire_gather(bi + NBUF, b)

            return 0

        lax.fori_loop(0, n_groups, group, 0)

        for b in range(NBUF):
            wait_scatter(b)

    return emb


def kernel(x, W):
    Bt, H = x.shape
    V, d = W.shape
    xt = jnp.transpose(x).astype(jnp.int32)
    out_phys = _build(Bt, H, V)(W, xt)
    out = jnp.transpose(out_phys, (2, 4, 0, 1, 3)).reshape(Bt, H, d)
    return out


# speedup vs baseline: 1.8623x; 1.2558x over previous
import functools
import math

import jax
import jax.numpy as jnp
from jax import lax
from jax.experimental import pallas as pl
from jax.experimental.pallas import tpu as pltpu
from jax.experimental.pallas import tpu_sc as plsc

D = 64
NC, NS = 2, 16
NW = NC * NS
SUB = 128
NBUF = 8
AHEAD = NBUF - 2


@functools.lru_cache(maxsize=None)
def _build_gather(B, V):
    assert B % (NW * SUB) == 0
    b_per_w = B // NW
    n_chunks = b_per_w // SUB
    idx_rows_w = n_chunks

    mesh = plsc.VectorSubcoreMesh(core_axis_name="c", subcore_axis_name="s")

    @functools.partial(
        pl.kernel,
        out_type=jax.ShapeDtypeStruct((B, D), jnp.float32),
        mesh=mesh,
        compiler_params=pltpu.CompilerParams(use_tc_tiling_on_sc=False),
        scratch_types=[
            pltpu.VMEM((idx_rows_w, SUB), jnp.int32),
            pltpu.VMEM((NBUF, SUB, D), jnp.float32),
            pltpu.SemaphoreType.DMA((NBUF,)),
            pltpu.SemaphoreType.DMA((NBUF,)),
        ],
    )
    def gather(w_hbm, x_hbm, out_hbm, idx_v, rows_v, gsem, osem):
        wid = lax.axis_index("s") * NC + lax.axis_index("c")
        base = wid * b_per_w

        pltpu.sync_copy(x_hbm.at[pl.ds(wid * idx_rows_w, idx_rows_w)], idx_v)

        def fire_gather(g, b):
            pltpu.async_copy(
                w_hbm.at[idx_v.at[g]], rows_v.at[b], gsem.at[b]
            )

        def wait_gather(b):
            pltpu.make_async_copy(
                w_hbm.at[idx_v.at[0]], rows_v.at[b], gsem.at[b]
            ).wait()

        def fire_scatter(g, b):
            pltpu.async_copy(
                rows_v.at[b],
                out_hbm.at[pl.ds(base + g * SUB, SUB)],
                osem.at[b],
            )

        def wait_scatter(b):
            pltpu.make_async_copy(
                rows_v.at[b], out_hbm.at[pl.ds(base, SUB)], osem.at[b]
            ).wait()

        for b in range(AHEAD):
            fire_gather(jnp.int32(b), b)

        def step(g, _):
            b = lax.rem(g, NBUF)
            wait_gather(b)
            fire_scatter(g, b)

            ga = g + AHEAD
            ba = lax.rem(ga, NBUF)

            @pl.when(ga < n_chunks)
            def _():
                @pl.when(g >= NBUF - AHEAD)
                def _():
                    wait_scatter(ba)

                fire_gather(ga, ba)

            return 0

        lax.fori_loop(0, n_chunks, step, 0)

        for g in range(n_chunks - (NBUF - AHEAD), n_chunks):
            wait_scatter(g % NBUF)

    return gather


@functools.lru_cache(maxsize=None)
def _build_transpose(B, H):
    n_btiles = B // H // SUB
    rows_per_tile = H * SUB * D // SUB
    scale = float(math.sqrt(D))

    def body(in_ref, out_ref):
        v = in_ref[0]
        x = v.reshape(SUB, H // 2, SUB)
        for h in range(H):
            sub = x[:, h // 2, (h % 2) * D:(h % 2) * D + D]
            y = jnp.transpose(sub) * scale
            out_ref[h, :, 0, :, :] = y.reshape(D // 8, 8, SUB)

    return pl.pallas_call(
        body,
        grid=(n_btiles,),
        in_specs=[
            pl.BlockSpec((1, rows_per_tile, SUB), lambda c: (c, 0, 0)),
        ],
        out_specs=pl.BlockSpec(
            (H, D // 8, 1, 8, SUB), lambda c: (0, 0, c, 0, 0)
        ),
        out_shape=jax.ShapeDtypeStruct(
            (H, D // 8, n_btiles, 8, SUB), jnp.float32
        ),
    )


def kernel(x, W):
    Bt, H = x.shape
    B = Bt * H
    V, d = W.shape
    xf = x.reshape(B // SUB, SUB).astype(jnp.int32)
    lin = _build_gather(B, V)(W, xf)
    lin3 = lin.reshape(B // (H * SUB), H * d, SUB)
    out_phys = _build_transpose(B, H)(lin3)
    out = jnp.transpose(out_phys, (2, 4, 0, 1, 3)).reshape(Bt, H, d)
    return out


# speedup vs baseline: 2.2932x; 1.2314x over previous
import functools
import math

import jax
import jax.numpy as jnp
from jax import lax
from jax.experimental import pallas as pl
from jax.experimental.pallas import tpu as pltpu
from jax.experimental.pallas import tpu_sc as plsc

D = 64
NC, NS = 2, 16
NW = NC * NS
SUB = 128
NBUF = 8
AHEAD = NBUF - 2


@functools.lru_cache(maxsize=None)
def _build_gather(B, V):
    assert B % (NW * SUB) == 0
    b_per_w = B // NW
    n_chunks = b_per_w // SUB
    idx_rows_w = n_chunks

    mesh = plsc.VectorSubcoreMesh(core_axis_name="c", subcore_axis_name="s")

    @functools.partial(
        pl.kernel,
        out_type=jax.ShapeDtypeStruct((B, D), jnp.float32),
        mesh=mesh,
        compiler_params=pltpu.CompilerParams(use_tc_tiling_on_sc=False),
        scratch_types=[
            pltpu.VMEM((idx_rows_w, SUB), jnp.int32),
            pltpu.VMEM((NBUF, SUB, D), jnp.float32),
            pltpu.SemaphoreType.DMA((NBUF,)),
            pltpu.SemaphoreType.DMA((NBUF,)),
        ],
    )
    def gather(w_hbm, x_hbm, out_hbm, idx_v, rows_v, gsem, osem):
        wid = lax.axis_index("s") * NC + lax.axis_index("c")
        base = wid * b_per_w

        pltpu.sync_copy(x_hbm.at[pl.ds(wid * idx_rows_w, idx_rows_w)], idx_v)

        def fire_gather(g, b):
            pltpu.async_copy(
                w_hbm.at[idx_v.at[g]], rows_v.at[b], gsem.at[b]
            )

        def wait_gather(b):
            pltpu.make_async_copy(
                w_hbm.at[idx_v.at[0]], rows_v.at[b], gsem.at[b]
            ).wait()

        def fire_scatter(g, b):
            pltpu.async_copy(
                rows_v.at[b],
                out_hbm.at[pl.ds(base + g * SUB, SUB)],
                osem.at[b],
            )

        def wait_scatter(b):
            pltpu.make_async_copy(
                rows_v.at[b], out_hbm.at[pl.ds(base, SUB)], osem.at[b]
            ).wait()

        for b in range(AHEAD):
            fire_gather(jnp.int32(b), b)

        def step(g, _):
            b = lax.rem(g, NBUF)
            wait_gather(b)
            fire_scatter(g, b)

            ga = g + AHEAD
            ba = lax.rem(ga, NBUF)

            @pl.when(ga < n_chunks)
            def _():
                @pl.when(g >= NBUF - AHEAD)
                def _():
                    wait_scatter(ba)

                fire_gather(ga, ba)

            return 0

        lax.fori_loop(0, n_chunks, step, 0)

        for g in range(n_chunks - (NBUF - AHEAD), n_chunks):
            wait_scatter(g % NBUF)

    return gather


@functools.lru_cache(maxsize=None)
def _build_w_relayout(V, d):
    BLK = 8192
    grid = (V + BLK - 1) // BLK

    def body(in_ref, out_ref):
        y = jnp.transpose(in_ref[...])
        y3 = y.reshape(BLK // 2, 2, d)
        out_ref[:, 0:d] = y3[:, 0, :]
        out_ref[:, d:2 * d] = y3[:, 1, :]

    return pl.pallas_call(
        body,
        grid=(grid,),
        in_specs=[pl.BlockSpec((d, BLK), lambda c: (0, c))],
        out_specs=pl.BlockSpec((BLK // 2, 2 * d), lambda c: (c, 0)),
        out_shape=jax.ShapeDtypeStruct((V // 2, 2 * d), jnp.float32),
    )


@functools.lru_cache(maxsize=None)
def _build_transpose(B, H):
    n_btiles = B // H // SUB
    rows_per_tile = H * SUB * D // SUB
    scale = float(math.sqrt(D))

    def body(in_ref, out_ref):
        v = in_ref[0]
        x = v.reshape(SUB, H // 2, SUB)
        for h in range(H):
            sub = x[:, h // 2, (h % 2) * D:(h % 2) * D + D]
            y = jnp.transpose(sub) * scale
            out_ref[h, :, 0, :, :] = y.reshape(D // 8, 8, SUB)

    return pl.pallas_call(
        body,
        grid=(n_btiles,),
        in_specs=[
            pl.BlockSpec((1, rows_per_tile, SUB), lambda c: (c, 0, 0)),
        ],
        out_specs=pl.BlockSpec(
            (H, D // 8, 1, 8, SUB), lambda c: (0, 0, c, 0, 0)
        ),
        out_shape=jax.ShapeDtypeStruct(
            (H, D // 8, n_btiles, 8, SUB), jnp.float32
        ),
    )


def kernel(x, W):
    Bt, H = x.shape
    B = Bt * H
    V, d = W.shape
    xf = x.reshape(B // SUB, SUB).astype(jnp.int32)
    w_lin = _build_w_relayout(V, d)(jnp.transpose(W))
    w_rows = w_lin.reshape(V, d)
    lin = _build_gather(B, V)(w_rows, xf)
    lin3 = lin.reshape(B // (H * SUB), H * d, SUB)
    out_phys = _build_transpose(B, H)(lin3)
    out = jnp.transpose(out_phys, (2, 4, 0, 1, 3)).reshape(Bt, H, d)
    return out
